# Initial kernel scaffold; baseline (speedup 1.0000x reference)
import jax
import jax.numpy as jnp
from jax.experimental import pallas as pl


def kernel(token_ids, weight):
    raise NotImplementedError("write your pallas kernel here")


# speedup vs baseline: 9.1827x; 9.1827x over previous
import functools

import jax
import jax.numpy as jnp
from jax import lax
from jax.experimental import pallas as pl
from jax.experimental.pallas import tpu as pltpu
from jax.experimental.pallas import tpu_sc as plsc

NUM_EMB = 100000
DIM = 128
TOTAL = 4096 * 200

NC = 2
NS = 16
NW = NC * NS
PER_W = TOTAL // NW
CHUNK = 128
NCHUNK = PER_W // CHUNK
NBUF = 4


def _sc_body(idx_hbm, table_hbm, out_hbm, idx_v, rows_v, gsem, osem):
    wid = lax.axis_index("s") * NC + lax.axis_index("c")
    row0 = wid * PER_W
    pltpu.sync_copy(idx_hbm.at[wid], idx_v)

    def step(i, carry):
        j = i * NBUF
        gathers = [
            pltpu.async_copy(table_hbm.at[idx_v.at[j + b]], rows_v.at[b], gsem)
            for b in range(NBUF)
        ]
        outs = []
        for b in range(NBUF):
            gathers[b].wait()
            outs.append(
                pltpu.async_copy(
                    rows_v.at[b],
                    out_hbm.at[pl.ds(row0 + (j + b) * CHUNK, CHUNK)],
                    osem,
                )
            )
        for b in range(NBUF):
            outs[b].wait()
        return carry

    lax.fori_loop(0, NCHUNK // NBUF, step, 0)


@jax.jit
def _embed(idx3, weight):
    mesh = plsc.VectorSubcoreMesh(core_axis_name="c", subcore_axis_name="s")
    k = functools.partial(
        pl.kernel,
        mesh=mesh,
        out_type=jax.ShapeDtypeStruct((TOTAL, DIM), jnp.float32),
        scratch_types=[
            pltpu.VMEM((NCHUNK, CHUNK), jnp.int32),
            pltpu.VMEM((NBUF, CHUNK, DIM), jnp.float32),
            pltpu.SemaphoreType.DMA,
            pltpu.SemaphoreType.DMA,
        ],
    )(_sc_body)
    return k(idx3, weight)


def kernel(token_ids, weight):
    idx3 = token_ids.astype(jnp.int32).reshape(NW, NCHUNK, CHUNK)
    out = _embed(idx3, weight)
    return out.reshape(token_ids.shape[0], token_ids.shape[1], DIM)
